# baseline (speedup 1.0000x reference)
import jax
import jax.numpy as jnp
from jax import lax
from jax.experimental import pallas as pl
from jax.experimental.pallas import tpu as pltpu
from jax.experimental.pallas import tpu_sc as plsc

_CH = 800
_GRP = 1
_ROWS_G = _CH * _GRP


def _make_body(per_w, nc):
    n_grp = per_w // _ROWS_G
    assert per_w % _ROWS_G == 0 and n_grp % 2 == 0

    def body(idx_hbm, table_hbm, out_hbm, idx_v, rows_v, gsem0, gsem1, wsem0, wsem1):
        c = lax.axis_index("c")
        s = lax.axis_index("s")
        wid = s * nc + c
        base = wid * per_w
        pltpu.sync_copy(idx_hbm.at[pl.ds(base, per_w)], idx_v)
        gsems = (gsem0, gsem1)
        wsems = (wsem0, wsem1)

        def fire_group(g, b):
            for j in range(_GRP):
                off = g * _ROWS_G + j * _CH
                pltpu.async_copy(
                    table_hbm.at[idx_v.at[pl.ds(off, _CH)]],
                    rows_v.at[b, pl.ds(j * _CH, _CH)],
                    gsems[b],
                )

        def drain_group(b):
            for j in range(_GRP):
                pltpu.make_async_copy(
                    table_hbm.at[idx_v.at[pl.ds(j * _CH, _CH)]],
                    rows_v.at[b, pl.ds(j * _CH, _CH)],
                    gsems[b],
                ).wait()

        def write_group(g, b):
            pltpu.async_copy(rows_v.at[b], out_hbm.at[pl.ds(base + g * _ROWS_G, _ROWS_G)], wsems[b])

        def wait_write(g, b):
            pltpu.make_async_copy(rows_v.at[b], out_hbm.at[pl.ds(base + g * _ROWS_G, _ROWS_G)], wsems[b]).wait()

        fire_group(0, 0)
        fire_group(1, 1)

        def outer(t, carry):
            g0 = 2 * t
            g1 = 2 * t + 1
            drain_group(0)
            write_group(g0, 0)
            drain_group(1)
            write_group(g1, 1)
            wait_write(g0, 0)
            fire_group((g0 + 2) % n_grp, 0)
            wait_write(g1, 1)
            fire_group((g1 + 2) % n_grp, 1)
            return carry

        lax.fori_loop(0, n_grp // 2, outer, 0)
        drain_group(0)
        drain_group(1)

    return body


def kernel(x, weight):
    b, h = x.shape
    _, d = weight.shape
    n = b * h
    idx = x.reshape(n).astype(jnp.int32)
    info = plsc.get_sparse_core_info()
    nw = info.num_cores * info.num_subcores
    per_w = n // nw
    out = pl.kernel(
        _make_body(per_w, info.num_cores),
        mesh=plsc.VectorSubcoreMesh(core_axis_name="c", subcore_axis_name="s"),
        compiler_params=pltpu.CompilerParams(use_tc_tiling_on_sc=False),
        out_type=jax.ShapeDtypeStruct((n, d), jnp.float32),
        scratch_types=[
            pltpu.VMEM((per_w,), jnp.int32),
            pltpu.VMEM((2, _ROWS_G, d), jnp.float32),
            pltpu.SemaphoreType.DMA,
            pltpu.SemaphoreType.DMA,
            pltpu.SemaphoreType.DMA,
            pltpu.SemaphoreType.DMA,
        ],
    )(idx, weight)
    return out.reshape(b, h, d)


# speedup vs baseline: 1.0258x; 1.0258x over previous
import jax
import jax.numpy as jnp
from jax import lax
from jax.experimental import pallas as pl
from jax.experimental.pallas import tpu as pltpu
from jax.experimental.pallas import tpu_sc as plsc

_CH = 400
_NBUF = 4


def _make_body(per_w, nc):
    n_grp = per_w // _CH
    assert per_w % _CH == 0 and n_grp >= _NBUF and n_grp % 2 == 0

    def body(idx_hbm, table_hbm, out_hbm, idx_v, rows_v, *sems):
        gsems, wsems = sems[:_NBUF], sems[_NBUF:]
        c = lax.axis_index("c")
        s = lax.axis_index("s")
        wid = s * nc + c
        base = wid * per_w
        pltpu.sync_copy(idx_hbm.at[pl.ds(base, per_w)], idx_v)

        def fire(g):
            b = g % _NBUF
            return pltpu.async_copy(
                table_hbm.at[idx_v.at[pl.ds(g * _CH, _CH)]],
                rows_v.at[b],
                gsems[b],
            )

        def write(g):
            b = g % _NBUF
            return pltpu.async_copy(
                rows_v.at[b],
                out_hbm.at[pl.ds(base + g * _CH, _CH)],
                wsems[b],
            )

        gathers = {0: fire(0), 1: fire(1)}
        writes = {}
        for g in range(n_grp):
            gathers.pop(g).wait()
            writes[g] = write(g)
            if g + 2 < n_grp:
                if g >= 2:
                    writes.pop(g - 2).wait()
                gathers[g + 2] = fire(g + 2)
        for g in sorted(writes):
            writes.pop(g).wait()

    return body


def kernel(x, weight):
    b, h = x.shape
    _, d = weight.shape
    n = b * h
    idx = x.reshape(n).astype(jnp.int32)
    info = plsc.get_sparse_core_info()
    nw = info.num_cores * info.num_subcores
    per_w = n // nw
    out = pl.kernel(
        _make_body(per_w, info.num_cores),
        mesh=plsc.VectorSubcoreMesh(core_axis_name="c", subcore_axis_name="s"),
        compiler_params=pltpu.CompilerParams(use_tc_tiling_on_sc=False),
        out_type=jax.ShapeDtypeStruct((n, d), jnp.float32),
        scratch_types=(
            [pltpu.VMEM((per_w,), jnp.int32),
             pltpu.VMEM((_NBUF, _CH, d), jnp.float32)]
            + [pltpu.SemaphoreType.DMA] * (2 * _NBUF)
        ),
    )(idx, weight)
    return out.reshape(b, h, d)
